# baseline (speedup 1.0000x reference)
import functools
import math

import jax
import jax.numpy as jnp
import numpy as np
from jax import lax
from jax.experimental import pallas as pl
from jax.experimental.pallas import tpu as pltpu
from jax.experimental.pallas import tpu_sc as plsc

N = 10000
E = 160000
MUL_IN = 32
MUL_S = 24
MUL_V = 8
HID = 64
F48 = MUL_S + MUL_V * 3

NC = 2
NS = 16
NW = NC * NS
CHUNK = 128
NCHUNK = 40
EPW = CHUNK * NCHUNK
EP = EPW * NW
NPAD = 10112
NBUF = 4
NCHT = EP // CHUNK
GCH0 = 20
GCH1 = 80 - GCH0
GMAX = max(GCH0, GCH1)
RPS = NPAD // NS

ALPHA = 1.0 / math.sqrt(float(MUL_IN))
Y0 = 1.0 / (2.0 * math.sqrt(math.pi))
C1 = math.sqrt(3.0 / (4.0 * math.pi))
W1024 = MUL_IN * (MUL_S + MUL_V)

_colidx = []
for _q in range(MUL_S):
    for _i in range(MUL_IN):
        _colidx.append(_i * MUL_S + _q)
for _j in range(MUL_V):
    for _i in range(MUL_IN):
        _colidx.append(MUL_IN * MUL_S + _i * MUL_V + _j)
_COLIDX = np.array(_colidx, dtype=np.int32)

_smat = np.zeros((W1024, F48), dtype=np.float32)
for _q in range(MUL_S):
    _smat[_q * MUL_IN:(_q + 1) * MUL_IN, _q] = 1.0
for _j in range(MUL_V):
    for _m in range(3):
        _smat[MUL_S * MUL_IN + _j * MUL_IN:
              MUL_S * MUL_IN + (_j + 1) * MUL_IN, MUL_S + 3 * _j + _m] = 1.0
_SMAT = np.array(_smat)

_ty = np.zeros((8, MUL_S), dtype=np.float32)
for _j in range(MUL_V):
    for _m, _c in enumerate((1, 2, 0)):
        _ty[_c, 3 * _j + _m] = 1.0
_TY = np.array(_ty)


def _sc_gather(table48, table16, src3, dst3):
    mesh = plsc.VectorSubcoreMesh(core_axis_name="c", subcore_axis_name="s")

    @functools.partial(
        pl.kernel,
        out_type=jax.ShapeDtypeStruct((EP, 128), jnp.float32),
        mesh=mesh,
        compiler_params=pltpu.CompilerParams(use_tc_tiling_on_sc=False),
        scratch_types=[
            pltpu.VMEM((GMAX, CHUNK), jnp.int32),
            pltpu.VMEM((GMAX, CHUNK), jnp.int32),
            pltpu.VMEM((NBUF, CHUNK, F48), jnp.float32),
            pltpu.VMEM((NBUF, CHUNK, 16), jnp.float32),
            [pltpu.SemaphoreType.DMA] * NBUF,
            [pltpu.SemaphoreType.DMA] * NBUF,
            [pltpu.SemaphoreType.DMA] * NBUF,
            [pltpu.SemaphoreType.DMA] * NBUF,
        ],
    )
    def k(t48, t16, src_h, dst_h, g_h, sidx, didx, r48, r16,
          sg48, sg16, so48, so16):
        c = lax.axis_index("c")
        s = lax.axis_index("s")
        cb = s * (GCH0 + GCH1) + c * GCH0
        nch = jnp.where(c == 0, GCH0, GCH1)
        pltpu.sync_copy(src_h.at[pl.ds(cb, GMAX)], sidx)
        pltpu.sync_copy(dst_h.at[pl.ds(cb, GMAX)], didx)

        def issue(j, b):
            pltpu.async_copy(t48.at[sidx.at[j]], r48.at[b], sg48[b])
            pltpu.async_copy(t16.at[didx.at[j]], r16.at[b], sg16[b])

        for b in range(NBUF):
            issue(b, b)

        def body(jj, carry):
            for b in range(NBUF):
                j = jj * NBUF + b
                off = (cb + j) * CHUNK
                pltpu.make_async_copy(g_h.at[pl.ds(0, CHUNK), pl.ds(0, F48)],
                                      r48.at[b], sg48[b]).wait()
                pltpu.make_async_copy(g_h.at[pl.ds(0, CHUNK), pl.ds(0, 16)],
                                      r16.at[b], sg16[b]).wait()
                pltpu.async_copy(r48.at[b],
                                 g_h.at[pl.ds(off, CHUNK), pl.ds(0, F48)],
                                 so48[b])
                pltpu.async_copy(r16.at[b],
                                 g_h.at[pl.ds(off, CHUNK), pl.ds(F48, 16)],
                                 so16[b])

                @pl.when(j + NBUF < nch)
                def _():
                    pltpu.make_async_copy(
                        r48.at[b], g_h.at[pl.ds(0, CHUNK), pl.ds(0, F48)],
                        so48[b]).wait()
                    pltpu.make_async_copy(
                        r16.at[b], g_h.at[pl.ds(0, CHUNK), pl.ds(F48, 16)],
                        so16[b]).wait()
                    issue(j + NBUF, b)
            return carry

        lax.fori_loop(0, nch // NBUF, body, 0)
        for b in range(NBUF):
            pltpu.make_async_copy(r48.at[b],
                                  g_h.at[pl.ds(0, CHUNK), pl.ds(0, F48)],
                                  so48[b]).wait()
            pltpu.make_async_copy(r16.at[b],
                                  g_h.at[pl.ds(0, CHUNK), pl.ds(F48, 16)],
                                  so16[b]).wait()

    return k(table48, table16, src3, dst3)


BLK = 2560


def _tc_dense_body(g_ref, wr1_ref, w2p_ref, s_ref, ty_ref, out_ref):
    g = g_ref[...]
    xs = g[:, :MUL_IN]
    ps = g[:, MUL_IN:MUL_IN + 3]
    pd = g[:, F48:F48 + 3]
    vec = pd - ps
    l2 = jnp.sum(vec * vec, axis=1, keepdims=True)
    length = jnp.maximum(jnp.sqrt(l2), 1e-8)
    vecn = vec / length
    h = length * wr1_ref[0:1, :]
    h = h * jax.nn.sigmoid(h)
    w = jnp.dot(h, w2p_ref[...], preferred_element_type=jnp.float32)
    x4 = jnp.concatenate([xs] * 4, axis=1)
    tile_x = jnp.concatenate([x4] * (W1024 // 128), axis=1)
    prod = w * tile_x
    big48 = jnp.dot(prod, s_ref[...], preferred_element_type=jnp.float32)
    y1t = jnp.dot(vecn, ty_ref[0:3, :], preferred_element_type=jnp.float32)
    m_s = (ALPHA * Y0) * big48[:, :MUL_S]
    m_v = (ALPHA * C1) * big48[:, MUL_S:] * y1t
    zpad = jnp.zeros((m_s.shape[0], 128 - F48), jnp.float32)
    out_ref[...] = jnp.concatenate([m_s, m_v, zpad], axis=1)


def _tc_dense(gfused, wr1p, w2p, smat, ty):
    return pl.pallas_call(
        _tc_dense_body,
        grid=(EP // BLK,),
        in_specs=[
            pl.BlockSpec((BLK, 128), lambda i: (i, 0)),
            pl.BlockSpec((8, HID), lambda i: (0, 0)),
            pl.BlockSpec((HID, W1024), lambda i: (0, 0)),
            pl.BlockSpec((W1024, F48), lambda i: (0, 0)),
            pl.BlockSpec((8, MUL_S), lambda i: (0, 0)),
        ],
        out_specs=pl.BlockSpec((BLK, 128), lambda i: (i, 0)),
        out_shape=jax.ShapeDtypeStruct((EP, 128), jnp.float32),
    )(gfused, wr1p, w2p, smat, ty)


def _sc_scatter(m48, dst3, zeros48):
    mesh = plsc.VectorSubcoreMesh(core_axis_name="c", subcore_axis_name="s")

    @functools.partial(
        pl.kernel,
        out_type=jax.ShapeDtypeStruct((NC, NPAD, F48), jnp.float32),
        mesh=mesh,
        compiler_params=pltpu.CompilerParams(use_tc_tiling_on_sc=False),
        scratch_types=[
            pltpu.VMEM((NCHUNK, CHUNK), jnp.int32),
            pltpu.VMEM((NBUF, CHUNK, F48), jnp.float32),
            pltpu.VMEM_SHARED((NPAD, F48), jnp.float32),
            [pltpu.SemaphoreType.DMA] * NBUF,
        ],
    )
    def k(m_h, dst_h, z_h, out_h, didx, rows, acc, sld):
        c = lax.axis_index("c")
        s = lax.axis_index("s")
        wid = s * NC + c
        base = wid * EPW
        pltpu.sync_copy(z_h.at[pl.ds(s * RPS, RPS)],
                        acc.at[pl.ds(s * RPS, RPS)])
        pltpu.sync_copy(dst_h.at[pl.ds(wid * NCHUNK, NCHUNK)], didx)
        plsc.subcore_barrier()

        for b in range(NBUF):
            pltpu.async_copy(
                m_h.at[pl.ds(base + b * CHUNK, CHUNK), pl.ds(0, F48)],
                rows.at[b], sld[b])

        def body(jj, carry):
            for b in range(NBUF):
                j = jj * NBUF + b
                pltpu.make_async_copy(
                    m_h.at[pl.ds(base, CHUNK), pl.ds(0, F48)],
                    rows.at[b], sld[b]).wait()
                pltpu.sync_copy(rows.at[b], acc.at[didx.at[j]], add=True)

                @pl.when(j + NBUF < NCHUNK)
                def _():
                    pltpu.async_copy(
                        m_h.at[pl.ds(base + (j + NBUF) * CHUNK, CHUNK),
                               pl.ds(0, F48)],
                        rows.at[b], sld[b])
            return carry

        lax.fori_loop(0, NCHUNK // NBUF, body, 0)
        plsc.subcore_barrier()
        pltpu.sync_copy(acc.at[pl.ds(s * RPS, RPS)],
                        out_h.at[c, pl.ds(s * RPS, RPS)])

    return k(m48, dst3, zeros48)


BLKN = 2000


def _tc_final_body(p_ref, ws_ref, wg_ref, wns_ref, out_ref):
    p = p_ref[0] + p_ref[1]
    s_in = p[:, :MUL_S]
    v48 = p[:, MUL_S:]
    sp = jnp.dot(s_in, ws_ref[...], preferred_element_type=jnp.float32)
    s = sp * jax.nn.sigmoid(sp)
    g = jax.nn.sigmoid(jnp.dot(s_in, wg_ref[...],
                               preferred_element_type=jnp.float32))
    ns = jnp.dot(v48, wns_ref[...], preferred_element_type=jnp.float32)
    out_ref[...] = s + g * ns


def _tc_final(partials, ws_s, wg_s, w48):
    return pl.pallas_call(
        _tc_final_body,
        grid=(N // BLKN,),
        in_specs=[
            pl.BlockSpec((NC, BLKN, F48), lambda i: (0, i, 0)),
            pl.BlockSpec((MUL_S, MUL_S), lambda i: (0, 0)),
            pl.BlockSpec((MUL_S, MUL_S), lambda i: (0, 0)),
            pl.BlockSpec((MUL_S, MUL_S), lambda i: (0, 0)),
        ],
        out_specs=pl.BlockSpec((BLKN, MUL_S), lambda i: (i, 0)),
        out_shape=jax.ShapeDtypeStruct((N, MUL_S), jnp.float32),
    )(partials, ws_s, wg_s, w48)


def kernel(x, edge_index, pos, W_r1, W_r2, Ws, Wns, Wg):
    src = edge_index[0]
    dst = edge_index[1]
    pad = EP - E
    src3 = jnp.concatenate(
        [src, jnp.zeros((pad + 40 * CHUNK,), jnp.int32)]
    ).reshape(NCHT + 40, CHUNK)
    dst3 = jnp.concatenate(
        [dst, jnp.full((pad,), N, jnp.int32),
         jnp.zeros((40 * CHUNK,), jnp.int32)]).reshape(NCHT + 40, CHUNK)
    znode = jnp.zeros((NPAD - N, 3), jnp.float32)
    table48 = jnp.concatenate(
        [x, pos, jnp.zeros((N, F48 - MUL_IN - 3), jnp.float32)], axis=1)
    table16 = jnp.concatenate([
        jnp.concatenate([pos, znode], axis=0),
        jnp.zeros((NPAD, 13), jnp.float32)], axis=1)

    gfused = _sc_gather(table48, table16, src3, dst3)

    scale = 1.0 / math.sqrt(float(HID))
    w2p = jnp.take(W_r2, _COLIDX, axis=1) * scale
    wr1p = jnp.concatenate([W_r1, jnp.zeros((7, HID), jnp.float32)], axis=0)
    ty = _TY

    m48 = _tc_dense(gfused, wr1p, w2p, _SMAT, ty)

    partials = _sc_scatter(m48, dst3, jnp.zeros((NPAD, F48), jnp.float32))

    ws_s = Ws / math.sqrt(float(MUL_S))
    wg_s = Wg / math.sqrt(float(MUL_S))
    w48 = jnp.kron(Wns, jnp.eye(3, dtype=jnp.float32)) / math.sqrt(float(MUL_V))
    return _tc_final(partials, ws_s, wg_s, w48)


# speedup vs baseline: 1.0262x; 1.0262x over previous
import functools
import math

import jax
import jax.numpy as jnp
import numpy as np
from jax import lax
from jax.experimental import pallas as pl
from jax.experimental.pallas import tpu as pltpu
from jax.experimental.pallas import tpu_sc as plsc

N = 10000
E = 160000
MUL_IN = 32
MUL_S = 24
MUL_V = 8
HID = 64
F48 = MUL_S + MUL_V * 3

NC = 2
NS = 16
NW = NC * NS
CHUNK = 128
NCHUNK = 40
EPW = CHUNK * NCHUNK
EP = EPW * NW
NPAD = 10112
NBUF = 4
NCHT = EP // CHUNK
GCH0 = 60
GCH1 = 80 - GCH0
GMAX = max(GCH0, GCH1)
RPS = NPAD // NS

ALPHA = 1.0 / math.sqrt(float(MUL_IN))
Y0 = 1.0 / (2.0 * math.sqrt(math.pi))
C1 = math.sqrt(3.0 / (4.0 * math.pi))
W1024 = MUL_IN * (MUL_S + MUL_V)

_colidx = []
for _q in range(MUL_S):
    for _i in range(MUL_IN):
        _colidx.append(_i * MUL_S + _q)
for _j in range(MUL_V):
    for _i in range(MUL_IN):
        _colidx.append(MUL_IN * MUL_S + _i * MUL_V + _j)
_COLIDX = np.array(_colidx, dtype=np.int32)

_smat = np.zeros((W1024, F48), dtype=np.float32)
for _q in range(MUL_S):
    _smat[_q * MUL_IN:(_q + 1) * MUL_IN, _q] = 1.0
for _j in range(MUL_V):
    for _m in range(3):
        _smat[MUL_S * MUL_IN + _j * MUL_IN:
              MUL_S * MUL_IN + (_j + 1) * MUL_IN, MUL_S + 3 * _j + _m] = 1.0
_SMAT = np.array(_smat)

_ty = np.zeros((8, MUL_S), dtype=np.float32)
for _j in range(MUL_V):
    for _m, _c in enumerate((1, 2, 0)):
        _ty[_c, 3 * _j + _m] = 1.0
_TY = np.array(_ty)


def _sc_gather(table48, table16, src3, dst3, half):
    mesh = plsc.VectorSubcoreMesh(core_axis_name="c", subcore_axis_name="s")
    hbase = half * (NCHT // 2)
    pc = 40
    g0 = 28
    g1 = 12

    @functools.partial(
        pl.kernel,
        out_type=jax.ShapeDtypeStruct((EP // 2, 128), jnp.float32),
        mesh=mesh,
        compiler_params=pltpu.CompilerParams(use_tc_tiling_on_sc=False),
        scratch_types=[
            pltpu.VMEM((GMAX // 2, CHUNK), jnp.int32),
            pltpu.VMEM((GMAX // 2, CHUNK), jnp.int32),
            pltpu.VMEM((NBUF, CHUNK, F48), jnp.float32),
            pltpu.VMEM((NBUF, CHUNK, 16), jnp.float32),
            [pltpu.SemaphoreType.DMA] * NBUF,
            [pltpu.SemaphoreType.DMA] * NBUF,
            [pltpu.SemaphoreType.DMA] * NBUF,
            [pltpu.SemaphoreType.DMA] * NBUF,
        ],
    )
    def k(t48, t16, src_h, dst_h, g_h, sidx, didx, r48, r16,
          sg48, sg16, so48, so16):
        c = lax.axis_index("c")
        s = lax.axis_index("s")
        cb = hbase + s * pc + c * g0
        nch = jnp.where(c == 0, g0, g1)
        pltpu.sync_copy(src_h.at[pl.ds(cb, GMAX // 2)], sidx)
        pltpu.sync_copy(dst_h.at[pl.ds(cb, GMAX // 2)], didx)

        def issue(j, b):
            pltpu.async_copy(t48.at[sidx.at[j]], r48.at[b], sg48[b])
            pltpu.async_copy(t16.at[didx.at[j]], r16.at[b], sg16[b])

        for b in range(NBUF):
            issue(b, b)

        def body(jj, carry):
            for b in range(NBUF):
                j = jj * NBUF + b
                off = (cb + j - hbase) * CHUNK
                pltpu.make_async_copy(g_h.at[pl.ds(0, CHUNK), pl.ds(0, F48)],
                                      r48.at[b], sg48[b]).wait()
                pltpu.make_async_copy(g_h.at[pl.ds(0, CHUNK), pl.ds(0, 16)],
                                      r16.at[b], sg16[b]).wait()
                pltpu.async_copy(r48.at[b],
                                 g_h.at[pl.ds(off, CHUNK), pl.ds(0, F48)],
                                 so48[b])
                pltpu.async_copy(r16.at[b],
                                 g_h.at[pl.ds(off, CHUNK), pl.ds(F48, 16)],
                                 so16[b])

                @pl.when(j + NBUF < nch)
                def _():
                    pltpu.make_async_copy(
                        r48.at[b], g_h.at[pl.ds(0, CHUNK), pl.ds(0, F48)],
                        so48[b]).wait()
                    pltpu.make_async_copy(
                        r16.at[b], g_h.at[pl.ds(0, CHUNK), pl.ds(F48, 16)],
                        so16[b]).wait()
                    issue(j + NBUF, b)
            return carry

        lax.fori_loop(0, nch // NBUF, body, 0)
        for b in range(NBUF):
            pltpu.make_async_copy(r48.at[b],
                                  g_h.at[pl.ds(0, CHUNK), pl.ds(0, F48)],
                                  so48[b]).wait()
            pltpu.make_async_copy(r16.at[b],
                                  g_h.at[pl.ds(0, CHUNK), pl.ds(F48, 16)],
                                  so16[b]).wait()

    return k(table48, table16, src3, dst3)


BLK = 2560


def _tc_dense_body(g_ref, wr1_ref, w2p_ref, s_ref, ty_ref, out_ref):
    g = g_ref[...]
    xs = g[:, :MUL_IN]
    ps = g[:, MUL_IN:MUL_IN + 3]
    pd = g[:, F48:F48 + 3]
    vec = pd - ps
    l2 = jnp.sum(vec * vec, axis=1, keepdims=True)
    length = jnp.maximum(jnp.sqrt(l2), 1e-8)
    vecn = vec / length
    h = length * wr1_ref[0:1, :]
    h = h * jax.nn.sigmoid(h)
    w = jnp.dot(h, w2p_ref[...], preferred_element_type=jnp.float32)
    x4 = jnp.concatenate([xs] * 4, axis=1)
    tile_x = jnp.concatenate([x4] * (W1024 // 128), axis=1)
    prod = w * tile_x
    big48 = jnp.dot(prod, s_ref[...], preferred_element_type=jnp.float32)
    y1t = jnp.dot(vecn, ty_ref[0:3, :], preferred_element_type=jnp.float32)
    m_s = (ALPHA * Y0) * big48[:, :MUL_S]
    m_v = (ALPHA * C1) * big48[:, MUL_S:] * y1t
    zpad = jnp.zeros((m_s.shape[0], 128 - F48), jnp.float32)
    out_ref[...] = jnp.concatenate([m_s, m_v, zpad], axis=1)


def _tc_dense(gfused, wr1p, w2p, smat, ty):
    return pl.pallas_call(
        _tc_dense_body,
        grid=(EP // 2 // BLK,),
        in_specs=[
            pl.BlockSpec((BLK, 128), lambda i: (i, 0)),
            pl.BlockSpec((8, HID), lambda i: (0, 0)),
            pl.BlockSpec((HID, W1024), lambda i: (0, 0)),
            pl.BlockSpec((W1024, F48), lambda i: (0, 0)),
            pl.BlockSpec((8, MUL_S), lambda i: (0, 0)),
        ],
        out_specs=pl.BlockSpec((BLK, 128), lambda i: (i, 0)),
        out_shape=jax.ShapeDtypeStruct((EP // 2, 128), jnp.float32),
    )(gfused, wr1p, w2p, smat, ty)


def _sc_scatter(m48, dst3, zeros48, half):
    mesh = plsc.VectorSubcoreMesh(core_axis_name="c", subcore_axis_name="s")
    hbase = half * (NCHT // 2)
    nchh = NCHUNK // 2

    @functools.partial(
        pl.kernel,
        out_type=jax.ShapeDtypeStruct((NC, NPAD, F48), jnp.float32),
        mesh=mesh,
        compiler_params=pltpu.CompilerParams(use_tc_tiling_on_sc=False),
        scratch_types=[
            pltpu.VMEM((NCHUNK // 2, CHUNK), jnp.int32),
            pltpu.VMEM((NBUF, CHUNK, F48), jnp.float32),
            pltpu.VMEM_SHARED((NPAD, F48), jnp.float32),
            [pltpu.SemaphoreType.DMA] * NBUF,
        ],
    )
    def k(m_h, dst_h, z_h, out_h, didx, rows, acc, sld):
        c = lax.axis_index("c")
        s = lax.axis_index("s")
        wid = s * NC + c
        base = wid * (EPW // 2)
        pltpu.sync_copy(z_h.at[pl.ds(s * RPS, RPS)],
                        acc.at[pl.ds(s * RPS, RPS)])
        pltpu.sync_copy(dst_h.at[pl.ds(hbase + wid * nchh, nchh)], didx)
        plsc.subcore_barrier()

        for b in range(NBUF):
            pltpu.async_copy(
                m_h.at[pl.ds(base + b * CHUNK, CHUNK), pl.ds(0, F48)],
                rows.at[b], sld[b])

        def body(jj, carry):
            for b in range(NBUF):
                j = jj * NBUF + b
                pltpu.make_async_copy(
                    m_h.at[pl.ds(base, CHUNK), pl.ds(0, F48)],
                    rows.at[b], sld[b]).wait()
                pltpu.sync_copy(rows.at[b], acc.at[didx.at[j]], add=True)

                @pl.when(j + NBUF < nchh)
                def _():
                    pltpu.async_copy(
                        m_h.at[pl.ds(base + (j + NBUF) * CHUNK, CHUNK),
                               pl.ds(0, F48)],
                        rows.at[b], sld[b])
            return carry

        lax.fori_loop(0, nchh // NBUF, body, 0)
        plsc.subcore_barrier()
        pltpu.sync_copy(acc.at[pl.ds(s * RPS, RPS)],
                        out_h.at[c, pl.ds(s * RPS, RPS)])

    return k(m48, dst3, zeros48)


BLKN = 2000


def _tc_final_body(p_ref, q_ref, ws_ref, wg_ref, wns_ref, out_ref):
    p = (p_ref[0] + p_ref[1]) + (q_ref[0] + q_ref[1])
    s_in = p[:, :MUL_S]
    v48 = p[:, MUL_S:]
    sp = jnp.dot(s_in, ws_ref[...], preferred_element_type=jnp.float32)
    s = sp * jax.nn.sigmoid(sp)
    g = jax.nn.sigmoid(jnp.dot(s_in, wg_ref[...],
                               preferred_element_type=jnp.float32))
    ns = jnp.dot(v48, wns_ref[...], preferred_element_type=jnp.float32)
    out_ref[...] = s + g * ns


def _tc_final(partials1, partials2, ws_s, wg_s, w48):
    return pl.pallas_call(
        _tc_final_body,
        grid=(N // BLKN,),
        in_specs=[
            pl.BlockSpec((NC, BLKN, F48), lambda i: (0, i, 0)),
            pl.BlockSpec((NC, BLKN, F48), lambda i: (0, i, 0)),
            pl.BlockSpec((MUL_S, MUL_S), lambda i: (0, 0)),
            pl.BlockSpec((MUL_S, MUL_S), lambda i: (0, 0)),
            pl.BlockSpec((MUL_S, MUL_S), lambda i: (0, 0)),
        ],
        out_specs=pl.BlockSpec((BLKN, MUL_S), lambda i: (i, 0)),
        out_shape=jax.ShapeDtypeStruct((N, MUL_S), jnp.float32),
    )(partials1, partials2, ws_s, wg_s, w48)


def kernel(x, edge_index, pos, W_r1, W_r2, Ws, Wns, Wg):
    src = edge_index[0]
    dst = edge_index[1]
    pad = EP - E
    src3 = jnp.concatenate(
        [src, jnp.zeros((pad + 40 * CHUNK,), jnp.int32)]
    ).reshape(NCHT + 40, CHUNK)
    dst3 = jnp.concatenate(
        [dst, jnp.full((pad,), N, jnp.int32),
         jnp.zeros((40 * CHUNK,), jnp.int32)]).reshape(NCHT + 40, CHUNK)
    znode = jnp.zeros((NPAD - N, 3), jnp.float32)
    table48 = jnp.concatenate(
        [x, pos, jnp.zeros((N, F48 - MUL_IN - 3), jnp.float32)], axis=1)
    table16 = jnp.concatenate([
        jnp.concatenate([pos, znode], axis=0),
        jnp.zeros((NPAD, 13), jnp.float32)], axis=1)

    g1 = _sc_gather(table48, table16, src3, dst3, 0)

    scale = 1.0 / math.sqrt(float(HID))
    w2p = jnp.take(W_r2, _COLIDX, axis=1) * scale
    wr1p = jnp.concatenate([W_r1, jnp.zeros((7, HID), jnp.float32)], axis=0)
    ty = _TY

    zeros = jnp.zeros((NPAD, F48), jnp.float32)
    m1 = _tc_dense(g1, wr1p, w2p, _SMAT, ty)
    g2 = _sc_gather(table48, table16, src3, dst3, 1)
    p1 = _sc_scatter(m1, dst3, zeros, 0)
    m2 = _tc_dense(g2, wr1p, w2p, _SMAT, ty)
    p2 = _sc_scatter(m2, dst3, zeros, 1)

    ws_s = Ws / math.sqrt(float(MUL_S))
    wg_s = Wg / math.sqrt(float(MUL_S))
    w48 = jnp.kron(Wns, jnp.eye(3, dtype=jnp.float32)) / math.sqrt(float(MUL_V))
    return _tc_final(p1, p2, ws_s, wg_s, w48)


# speedup vs baseline: 1.0350x; 1.0086x over previous
import functools
import math

import jax
import jax.numpy as jnp
import numpy as np
from jax import lax
from jax.experimental import pallas as pl
from jax.experimental.pallas import tpu as pltpu
from jax.experimental.pallas import tpu_sc as plsc

N = 10000
E = 160000
MUL_IN = 32
MUL_S = 24
MUL_V = 8
HID = 64
F48 = MUL_S + MUL_V * 3

NC = 2
NS = 16
NW = NC * NS
CHUNK = 128
NCHUNK = 40
EPW = CHUNK * NCHUNK
EP = EPW * NW
NPAD = 10112
NBUF = 4
NCHT = EP // CHUNK
GCH0 = 60
GCH1 = 80 - GCH0
GMAX = max(GCH0, GCH1)
RPS = NPAD // NS

ALPHA = 1.0 / math.sqrt(float(MUL_IN))
Y0 = 1.0 / (2.0 * math.sqrt(math.pi))
C1 = math.sqrt(3.0 / (4.0 * math.pi))
W1024 = MUL_IN * (MUL_S + MUL_V)

_colidx = []
for _q in range(MUL_S):
    for _i in range(MUL_IN):
        _colidx.append(_i * MUL_S + _q)
for _j in range(MUL_V):
    for _i in range(MUL_IN):
        _colidx.append(MUL_IN * MUL_S + _i * MUL_V + _j)
_COLIDX = np.array(_colidx, dtype=np.int32)

_smat = np.zeros((W1024, F48), dtype=np.float32)
for _q in range(MUL_S):
    _smat[_q * MUL_IN:(_q + 1) * MUL_IN, _q] = 1.0
for _j in range(MUL_V):
    for _m in range(3):
        _smat[MUL_S * MUL_IN + _j * MUL_IN:
              MUL_S * MUL_IN + (_j + 1) * MUL_IN, MUL_S + 3 * _j + _m] = 1.0
_SMAT = np.array(_smat)

_ty = np.zeros((8, MUL_S), dtype=np.float32)
for _j in range(MUL_V):
    for _m, _c in enumerate((1, 2, 0)):
        _ty[_c, 3 * _j + _m] = 1.0
_TY = np.array(_ty)


def _sc_gather(table48, table16, src3, dst3, half):
    mesh = plsc.VectorSubcoreMesh(core_axis_name="c", subcore_axis_name="s")
    hbase = half * (NCHT // 2)
    pc = 40
    g0 = 20
    g1 = 20

    @functools.partial(
        pl.kernel,
        out_type=jax.ShapeDtypeStruct((EP // 2, 128), jnp.float32),
        mesh=mesh,
        compiler_params=pltpu.CompilerParams(use_tc_tiling_on_sc=False),
        scratch_types=[
            pltpu.VMEM((GMAX // 2, CHUNK), jnp.int32),
            pltpu.VMEM((GMAX // 2, CHUNK), jnp.int32),
            pltpu.VMEM((NBUF, CHUNK, F48), jnp.float32),
            pltpu.VMEM((NBUF, CHUNK, 16), jnp.float32),
            [pltpu.SemaphoreType.DMA] * NBUF,
            [pltpu.SemaphoreType.DMA] * NBUF,
            [pltpu.SemaphoreType.DMA] * NBUF,
            [pltpu.SemaphoreType.DMA] * NBUF,
        ],
    )
    def k(t48, t16, src_h, dst_h, g_h, sidx, didx, r48, r16,
          sg48, sg16, so48, so16):
        c = lax.axis_index("c")
        s = lax.axis_index("s")
        cb = hbase + s * pc + c * g0
        nch = jnp.where(c == 0, g0, g1)
        pltpu.sync_copy(src_h.at[pl.ds(cb, GMAX // 2)], sidx)
        pltpu.sync_copy(dst_h.at[pl.ds(cb, GMAX // 2)], didx)

        def issue(j, b):
            pltpu.async_copy(t48.at[sidx.at[j]], r48.at[b], sg48[b])
            pltpu.async_copy(t16.at[didx.at[j]], r16.at[b], sg16[b])

        for b in range(NBUF):
            issue(b, b)

        def body(jj, carry):
            for b in range(NBUF):
                j = jj * NBUF + b
                off = (cb + j - hbase) * CHUNK
                pltpu.make_async_copy(g_h.at[pl.ds(0, CHUNK), pl.ds(0, F48)],
                                      r48.at[b], sg48[b]).wait()
                pltpu.make_async_copy(g_h.at[pl.ds(0, CHUNK), pl.ds(0, 16)],
                                      r16.at[b], sg16[b]).wait()
                pltpu.async_copy(r48.at[b],
                                 g_h.at[pl.ds(off, CHUNK), pl.ds(0, F48)],
                                 so48[b])
                pltpu.async_copy(r16.at[b],
                                 g_h.at[pl.ds(off, CHUNK), pl.ds(F48, 16)],
                                 so16[b])

                @pl.when(j + NBUF < nch)
                def _():
                    pltpu.make_async_copy(
                        r48.at[b], g_h.at[pl.ds(0, CHUNK), pl.ds(0, F48)],
                        so48[b]).wait()
                    pltpu.make_async_copy(
                        r16.at[b], g_h.at[pl.ds(0, CHUNK), pl.ds(F48, 16)],
                        so16[b]).wait()
                    issue(j + NBUF, b)
            return carry

        lax.fori_loop(0, nch // NBUF, body, 0)
        for b in range(NBUF):
            pltpu.make_async_copy(r48.at[b],
                                  g_h.at[pl.ds(0, CHUNK), pl.ds(0, F48)],
                                  so48[b]).wait()
            pltpu.make_async_copy(r16.at[b],
                                  g_h.at[pl.ds(0, CHUNK), pl.ds(F48, 16)],
                                  so16[b]).wait()

    return k(table48, table16, src3, dst3)


BLK = 2560


def _tc_dense_body(g_ref, wr1_ref, w2p_ref, s_ref, ty_ref, out_ref):
    g = g_ref[...]
    xs = g[:, :MUL_IN]
    ps = g[:, MUL_IN:MUL_IN + 3]
    pd = g[:, F48:F48 + 3]
    vec = pd - ps
    l2 = jnp.sum(vec * vec, axis=1, keepdims=True)
    length = jnp.maximum(jnp.sqrt(l2), 1e-8)
    vecn = vec / length
    h = length * wr1_ref[0:1, :]
    h = h * jax.nn.sigmoid(h)
    w = jnp.dot(h, w2p_ref[...], preferred_element_type=jnp.float32)
    x4 = jnp.concatenate([xs] * 4, axis=1)
    tile_x = jnp.concatenate([x4] * (W1024 // 128), axis=1)
    prod = w * tile_x
    big48 = jnp.dot(prod, s_ref[...], preferred_element_type=jnp.float32)
    y1t = jnp.dot(vecn, ty_ref[0:3, :], preferred_element_type=jnp.float32)
    m_s = (ALPHA * Y0) * big48[:, :MUL_S]
    m_v = (ALPHA * C1) * big48[:, MUL_S:] * y1t
    zpad = jnp.zeros((m_s.shape[0], 128 - F48), jnp.float32)
    out_ref[...] = jnp.concatenate([m_s, m_v, zpad], axis=1)


def _tc_dense(gfused, wr1p, w2p, smat, ty):
    return pl.pallas_call(
        _tc_dense_body,
        grid=(EP // 2 // BLK,),
        in_specs=[
            pl.BlockSpec((BLK, 128), lambda i: (i, 0)),
            pl.BlockSpec((8, HID), lambda i: (0, 0)),
            pl.BlockSpec((HID, W1024), lambda i: (0, 0)),
            pl.BlockSpec((W1024, F48), lambda i: (0, 0)),
            pl.BlockSpec((8, MUL_S), lambda i: (0, 0)),
        ],
        out_specs=pl.BlockSpec((BLK, 128), lambda i: (i, 0)),
        out_shape=jax.ShapeDtypeStruct((EP // 2, 128), jnp.float32),
    )(gfused, wr1p, w2p, smat, ty)


def _sc_scatter(m48, dst3, zeros48, half):
    mesh = plsc.VectorSubcoreMesh(core_axis_name="c", subcore_axis_name="s")
    hbase = half * (NCHT // 2)
    nchh = NCHUNK // 2

    @functools.partial(
        pl.kernel,
        out_type=jax.ShapeDtypeStruct((NC, NPAD, F48), jnp.float32),
        mesh=mesh,
        compiler_params=pltpu.CompilerParams(use_tc_tiling_on_sc=False),
        scratch_types=[
            pltpu.VMEM((NCHUNK // 2, CHUNK), jnp.int32),
            pltpu.VMEM((NBUF, CHUNK, F48), jnp.float32),
            pltpu.VMEM_SHARED((NPAD, F48), jnp.float32),
            [pltpu.SemaphoreType.DMA] * NBUF,
        ],
    )
    def k(m_h, dst_h, z_h, out_h, didx, rows, acc, sld):
        c = lax.axis_index("c")
        s = lax.axis_index("s")
        wid = s * NC + c
        base = wid * (EPW // 2)
        pltpu.sync_copy(z_h.at[pl.ds(s * RPS, RPS)],
                        acc.at[pl.ds(s * RPS, RPS)])
        pltpu.sync_copy(dst_h.at[pl.ds(hbase + wid * nchh, nchh)], didx)
        plsc.subcore_barrier()

        for b in range(NBUF):
            pltpu.async_copy(
                m_h.at[pl.ds(base + b * CHUNK, CHUNK), pl.ds(0, F48)],
                rows.at[b], sld[b])

        def body(jj, carry):
            for b in range(NBUF):
                j = jj * NBUF + b
                pltpu.make_async_copy(
                    m_h.at[pl.ds(base, CHUNK), pl.ds(0, F48)],
                    rows.at[b], sld[b]).wait()
                pltpu.sync_copy(rows.at[b], acc.at[didx.at[j]], add=True)

                @pl.when(j + NBUF < nchh)
                def _():
                    pltpu.async_copy(
                        m_h.at[pl.ds(base + (j + NBUF) * CHUNK, CHUNK),
                               pl.ds(0, F48)],
                        rows.at[b], sld[b])
            return carry

        lax.fori_loop(0, nchh // NBUF, body, 0)
        plsc.subcore_barrier()
        pltpu.sync_copy(acc.at[pl.ds(s * RPS, RPS)],
                        out_h.at[c, pl.ds(s * RPS, RPS)])

    return k(m48, dst3, zeros48)


BLKN = 2000


def _tc_final_body(p_ref, q_ref, ws_ref, wg_ref, wns_ref, out_ref):
    p = (p_ref[0] + p_ref[1]) + (q_ref[0] + q_ref[1])
    s_in = p[:, :MUL_S]
    v48 = p[:, MUL_S:]
    sp = jnp.dot(s_in, ws_ref[...], preferred_element_type=jnp.float32)
    s = sp * jax.nn.sigmoid(sp)
    g = jax.nn.sigmoid(jnp.dot(s_in, wg_ref[...],
                               preferred_element_type=jnp.float32))
    ns = jnp.dot(v48, wns_ref[...], preferred_element_type=jnp.float32)
    out_ref[...] = s + g * ns


def _tc_final(partials1, partials2, ws_s, wg_s, w48):
    return pl.pallas_call(
        _tc_final_body,
        grid=(N // BLKN,),
        in_specs=[
            pl.BlockSpec((NC, BLKN, F48), lambda i: (0, i, 0)),
            pl.BlockSpec((NC, BLKN, F48), lambda i: (0, i, 0)),
            pl.BlockSpec((MUL_S, MUL_S), lambda i: (0, 0)),
            pl.BlockSpec((MUL_S, MUL_S), lambda i: (0, 0)),
            pl.BlockSpec((MUL_S, MUL_S), lambda i: (0, 0)),
        ],
        out_specs=pl.BlockSpec((BLKN, MUL_S), lambda i: (i, 0)),
        out_shape=jax.ShapeDtypeStruct((N, MUL_S), jnp.float32),
    )(partials1, partials2, ws_s, wg_s, w48)


def kernel(x, edge_index, pos, W_r1, W_r2, Ws, Wns, Wg):
    src = edge_index[0]
    dst = edge_index[1]
    pad = EP - E
    src3 = jnp.concatenate(
        [src, jnp.zeros((pad + 40 * CHUNK,), jnp.int32)]
    ).reshape(NCHT + 40, CHUNK)
    dst3 = jnp.concatenate(
        [dst, jnp.full((pad,), N, jnp.int32),
         jnp.zeros((40 * CHUNK,), jnp.int32)]).reshape(NCHT + 40, CHUNK)
    znode = jnp.zeros((NPAD - N, 3), jnp.float32)
    table48 = jnp.concatenate(
        [x, pos, jnp.zeros((N, F48 - MUL_IN - 3), jnp.float32)], axis=1)
    table16 = jnp.concatenate([
        jnp.concatenate([pos, znode], axis=0),
        jnp.zeros((NPAD, 13), jnp.float32)], axis=1)

    g1 = _sc_gather(table48, table16, src3, dst3, 0)

    scale = 1.0 / math.sqrt(float(HID))
    w2p = jnp.take(W_r2, _COLIDX, axis=1) * scale
    wr1p = jnp.concatenate([W_r1, jnp.zeros((7, HID), jnp.float32)], axis=0)
    ty = _TY

    zeros = jnp.zeros((NPAD, F48), jnp.float32)
    m1 = _tc_dense(g1, wr1p, w2p, _SMAT, ty)
    g2 = _sc_gather(table48, table16, src3, dst3, 1)
    p1 = _sc_scatter(m1, dst3, zeros, 0)
    m2 = _tc_dense(g2, wr1p, w2p, _SMAT, ty)
    p2 = _sc_scatter(m2, dst3, zeros, 1)

    ws_s = Ws / math.sqrt(float(MUL_S))
    wg_s = Wg / math.sqrt(float(MUL_S))
    w48 = jnp.kron(Wns, jnp.eye(3, dtype=jnp.float32)) / math.sqrt(float(MUL_V))
    return _tc_final(p1, p2, ws_s, wg_s, w48)
